# baseline (speedup 1.0000x reference)
import functools

import jax
import jax.numpy as jnp
from jax import lax
from jax.experimental import pallas as pl
from jax.experimental.pallas import tpu as pltpu
from jax.experimental.pallas import tpu_sc as plsc

SPACE_SIZE = 100000
OUT_DIM = 16
FIELD = 26
BATCH = 16384

_info = plsc.get_sparse_core_info()
_NC, _NS = _info.num_cores, _info.num_subcores
_NW = _NC * _NS
_B_PER_W = BATCH // _NW


@functools.partial(
    pl.kernel,
    mesh=plsc.VectorSubcoreMesh(core_axis_name="c", subcore_axis_name="s"),
    out_type=jax.ShapeDtypeStruct((BATCH, OUT_DIM, FIELD), jnp.float32),
    scratch_types=[
        pltpu.VMEM((_B_PER_W,), jnp.int32),
        pltpu.SemaphoreType.DMA,
    ],
)
def _lookup(table_hbm, idx_hbm, out_hbm, idx_vm, sem):
    wid = lax.axis_index("s") * _NC + lax.axis_index("c")
    base = wid * _B_PER_W
    pltpu.sync_copy(idx_hbm.at[pl.ds(base, _B_PER_W)], idx_vm)

    def body(g, carry):
        v = idx_vm[pl.ds(g * 16, 16)]
        row0 = base + g * 16
        copies = [
            pltpu.async_copy(table_hbm.at[v[k]], out_hbm.at[row0 + k], sem)
            for k in range(16)
        ]
        for c in copies:
            c.wait()
        return carry

    lax.fori_loop(0, _B_PER_W // 16, body, 0)


def kernel(x, ker):
    out = _lookup(ker, x.astype(jnp.int32))
    return out[..., None]


# speedup vs baseline: 6.4318x; 6.4318x over previous
import functools

import jax
import jax.numpy as jnp
from jax import lax
from jax.experimental import pallas as pl
from jax.experimental.pallas import tpu as pltpu
from jax.experimental.pallas import tpu_sc as plsc

SPACE_SIZE = 100000
OUT_DIM = 16
FIELD = 26
BATCH = 16384

_info = plsc.get_sparse_core_info()
_NC, _NS = _info.num_cores, _info.num_subcores
_NW = _NC * _NS
_B_PER_W = BATCH // _NW
_G = 16
_NGRP = _B_PER_W // _G


@functools.partial(
    pl.kernel,
    mesh=plsc.VectorSubcoreMesh(core_axis_name="c", subcore_axis_name="s"),
    out_type=jax.ShapeDtypeStruct((BATCH, OUT_DIM, FIELD), jnp.float32),
    scratch_types=[
        pltpu.VMEM((_B_PER_W,), jnp.int32),
        pltpu.VMEM((2, _G, OUT_DIM, FIELD), jnp.float32),
        pltpu.SemaphoreType.DMA,
        pltpu.SemaphoreType.DMA,
    ],
)
def _lookup(table_hbm, idx_hbm, out_hbm, idx_vm, rows_v, rsem, wsem):
    wid = lax.axis_index("s") * _NC + lax.axis_index("c")
    base = wid * _B_PER_W
    pltpu.sync_copy(idx_hbm.at[pl.ds(base, _B_PER_W)], idx_vm)

    def do_group(g, buf):
        v = idx_vm[pl.ds(g * _G, _G)]
        reads = [
            pltpu.async_copy(table_hbm.at[v[k]], rows_v.at[buf, k], rsem)
            for k in range(_G)
        ]
        for r in reads:
            r.wait()
        pltpu.async_copy(
            rows_v.at[buf], out_hbm.at[pl.ds(base + g * _G, _G)], wsem
        )

    def wait_one_write(g):
        pltpu.make_async_copy(
            rows_v.at[0], out_hbm.at[pl.ds(base, _G)], wsem
        ).wait()

    do_group(0, 0)
    do_group(1, 1)

    def body(g, carry):
        wait_one_write(g)
        do_group(g, lax.rem(g, 2))
        return carry

    lax.fori_loop(2, _NGRP, body, 0)
    wait_one_write(_NGRP - 1)
    wait_one_write(_NGRP)


def kernel(x, ker):
    out = _lookup(ker, x.astype(jnp.int32))
    return out[..., None]


# speedup vs baseline: 17.5464x; 2.7281x over previous
import functools

import jax
import jax.numpy as jnp
from jax import lax
from jax.experimental import pallas as pl
from jax.experimental.pallas import tpu as pltpu
from jax.experimental.pallas import tpu_sc as plsc

SPACE_SIZE = 100000
OUT_DIM = 16
FIELD = 26
BATCH = 16384

_C = OUT_DIM * FIELD
_info = plsc.get_sparse_core_info()
_NC, _NS = _info.num_cores, _info.num_subcores
_NW = _NC * _NS
_C_PER_W = _C // _NW
_OCHUNK = 2048


@functools.partial(
    pl.kernel,
    mesh=plsc.VectorSubcoreMesh(core_axis_name="c", subcore_axis_name="s"),
    out_type=jax.ShapeDtypeStruct((_C, BATCH), jnp.float32),
    scratch_types=[
        pltpu.VMEM((BATCH,), jnp.int32),
        pltpu.VMEM((SPACE_SIZE,), jnp.float32),
        pltpu.VMEM((2, _OCHUNK), jnp.float32),
        pltpu.SemaphoreType.DMA,
    ],
    compiler_params=pltpu.CompilerParams(needs_layout_passes=False),
)
def _lookup(table_hbm, idx_hbm, out_hbm, idx_v, col_v, ob, wsem):
    wid = lax.axis_index("s") * _NC + lax.axis_index("c")
    pltpu.sync_copy(idx_hbm, idx_v)

    nchunk = BATCH // _OCHUNK
    for ci in range(_C_PER_W):
        col = wid * _C_PER_W + ci
        pltpu.sync_copy(table_hbm.at[col], col_v)
        for ch in range(nchunk):
            buf = ch % 2

            def grp(g, carry, _ch=ch, _buf=buf):
                i16 = idx_v[pl.ds(_ch * _OCHUNK + g * 16, 16)]
                ob[_buf, pl.ds(g * 16, 16)] = plsc.load_gather(col_v, [i16])
                return carry

            if ci > 0 or ch > 1:
                pltpu.make_async_copy(
                    ob.at[buf], out_hbm.at[col, pl.ds(0, _OCHUNK)], wsem
                ).wait()
            lax.fori_loop(0, _OCHUNK // 16, grp, 0)
            pltpu.async_copy(
                ob.at[buf], out_hbm.at[col, pl.ds(ch * _OCHUNK, _OCHUNK)], wsem
            )
    pltpu.make_async_copy(
        ob.at[0], out_hbm.at[0, pl.ds(0, _OCHUNK)], wsem
    ).wait()
    pltpu.make_async_copy(
        ob.at[1], out_hbm.at[0, pl.ds(0, _OCHUNK)], wsem
    ).wait()


def kernel(x, ker):
    kt = jnp.transpose(ker, (2, 1, 0)).reshape(_C, SPACE_SIZE)
    out = _lookup(kt, x.astype(jnp.int32))
    return jnp.transpose(out.reshape(OUT_DIM, FIELD, BATCH), (2, 0, 1))[..., None]


# speedup vs baseline: 19.7683x; 1.1266x over previous
import functools

import jax
import jax.numpy as jnp
from jax import lax
from jax.experimental import pallas as pl
from jax.experimental.pallas import tpu as pltpu
from jax.experimental.pallas import tpu_sc as plsc

SPACE_SIZE = 100000
OUT_DIM = 16
FIELD = 26
BATCH = 16384

_C = OUT_DIM * FIELD
_info = plsc.get_sparse_core_info()
_NC, _NS = _info.num_cores, _info.num_subcores
_NW = _NC * _NS
_C_PER_W = _C // _NW
_OCHUNK = 2048


@functools.partial(
    pl.kernel,
    mesh=plsc.VectorSubcoreMesh(core_axis_name="c", subcore_axis_name="s"),
    out_type=jax.ShapeDtypeStruct((_C, BATCH), jnp.float32),
    scratch_types=[
        pltpu.VMEM((BATCH,), jnp.int32),
        pltpu.VMEM((SPACE_SIZE,), jnp.float32),
        pltpu.VMEM((2, _OCHUNK), jnp.float32),
        pltpu.SemaphoreType.DMA,
    ],
    compiler_params=pltpu.CompilerParams(needs_layout_passes=False),
)
def _lookup(table_hbm, idx_hbm, out_hbm, idx_v, col_v, ob, wsem):
    wid = lax.axis_index("s") * _NC + lax.axis_index("c")
    pltpu.sync_copy(idx_hbm, idx_v)

    nchunk = BATCH // _OCHUNK
    for ci in range(_C_PER_W):
        col = wid * _C_PER_W + ci
        pltpu.sync_copy(table_hbm.at[col], col_v)
        for ch in range(nchunk):
            buf = ch % 2

            def grp(g, carry, _ch=ch, _buf=buf):
                i16 = idx_v[pl.ds(_ch * _OCHUNK + g * 16, 16)]
                ob[_buf, pl.ds(g * 16, 16)] = plsc.load_gather(col_v, [i16])
                return carry

            if ci > 0 or ch > 1:
                pltpu.make_async_copy(
                    ob.at[buf], out_hbm.at[col, pl.ds(0, _OCHUNK)], wsem
                ).wait()
            lax.fori_loop(0, _OCHUNK // 16, grp, 0)
            pltpu.async_copy(
                ob.at[buf], out_hbm.at[col, pl.ds(ch * _OCHUNK, _OCHUNK)], wsem
            )
    pltpu.make_async_copy(
        ob.at[0], out_hbm.at[0, pl.ds(0, _OCHUNK)], wsem
    ).wait()
    pltpu.make_async_copy(
        ob.at[1], out_hbm.at[0, pl.ds(0, _OCHUNK)], wsem
    ).wait()


def kernel(x, ker):
    kt = jnp.transpose(ker, (2, 1, 0)).reshape(_C, SPACE_SIZE)
    out = _lookup(kt, x.astype(jnp.int32))
    return jnp.transpose(out.reshape(FIELD, OUT_DIM, BATCH), (2, 1, 0))[..., None]
